# baseline (speedup 1.0000x reference)
import functools

import jax
import jax.numpy as jnp
from jax import lax
from jax.experimental import pallas as pl
from jax.experimental.pallas import tpu as pltpu
from jax.experimental.pallas import tpu_sc as plsc

N = 10000
D = 128
H = 16
C = 40
E = 320000
NP = 10240
NC = 2
NS = 16
NW = NC * NS
K = 128
EP = 327680
EW = EP // NW
NCH = EW // K
RPT = NP // NS

_mesh = plsc.VectorSubcoreMesh(core_axis_name="c", subcore_axis_name="s")


@functools.partial(
    pl.kernel,
    out_type=jax.ShapeDtypeStruct((NC, NP), jnp.float32),
    mesh=_mesh,
    scratch_types=[
        pltpu.VMEM((NCH, K), jnp.int32),
        pltpu.VMEM((K,), jnp.float32),
        pltpu.VMEM((RPT,), jnp.float32),
        pltpu.VMEM_SHARED((NP,), jnp.float32),
    ],
)
def _deg_kernel(dst_hbm, out_hbm, didx_v, ones_v, zero_v, deg_sh):
    c = lax.axis_index("c")
    s = lax.axis_index("s")
    w = c * NS + s
    for i in range(K // 16):
        ones_v[pl.ds(i * 16, 16)] = jnp.ones((16,), jnp.float32)
    for i in range(RPT // 16):
        zero_v[pl.ds(i * 16, 16)] = jnp.zeros((16,), jnp.float32)
    pltpu.sync_copy(zero_v, deg_sh.at[pl.ds(s * RPT, RPT)])
    plsc.subcore_barrier()
    pltpu.sync_copy(dst_hbm.at[w], didx_v)

    def chunk(j, carry):
        pltpu.sync_copy(ones_v, deg_sh.at[didx_v.at[j]], add=True)
        return carry

    lax.fori_loop(0, NCH, chunk, 0)
    plsc.subcore_barrier()
    pltpu.sync_copy(deg_sh.at[pl.ds(s * RPT, RPT)],
                    out_hbm.at[c, pl.ds(s * RPT, RPT)])


@functools.partial(
    pl.kernel,
    out_type=jax.ShapeDtypeStruct((NC, NP, H), jnp.float32),
    mesh=_mesh,
    scratch_types=[
        pltpu.VMEM((NCH, K), jnp.int32),
        pltpu.VMEM((NCH, K), jnp.int32),
        pltpu.VMEM((K, H), jnp.float32),
        pltpu.VMEM((K, H), jnp.float32),
        pltpu.VMEM_SHARED((NP, H), jnp.float32),
        pltpu.SemaphoreType.DMA,
        pltpu.SemaphoreType.DMA,
    ],
    compiler_params=pltpu.CompilerParams(use_tc_tiling_on_sc=False),
)
def _scatter_kernel(g_hbm, src_hbm, dst_hbm, out_hbm, sidx_v, didx_v, rows0_v,
                    rows1_v, acc_sh, sem0, sem1):
    c = lax.axis_index("c")
    s = lax.axis_index("s")
    w = c * NS + s
    for i in range(K):
        rows0_v[i, :] = jnp.zeros((H,), jnp.float32)

    for t in range(RPT // K):
        pltpu.sync_copy(rows0_v, acc_sh.at[pl.ds(s * RPT + t * K, K)])
    plsc.subcore_barrier()
    pltpu.sync_copy(src_hbm.at[w], sidx_v)
    pltpu.sync_copy(dst_hbm.at[w], didx_v)

    def gather(j, rows, sem):
        pltpu.async_copy(g_hbm.at[sidx_v.at[j]], rows, sem)

    def gwait(rows, sem):
        pltpu.make_async_copy(g_hbm.at[sidx_v.at[0]], rows, sem).wait()

    def scat(j, rows):
        pltpu.sync_copy(rows, acc_sh.at[didx_v.at[j]], add=True)

    gather(0, rows0_v, sem0)

    def pair(i, carry):
        j0 = 2 * i
        gather(j0 + 1, rows1_v, sem1)
        gwait(rows0_v, sem0)
        scat(j0, rows0_v)
        gather(j0 + 2, rows0_v, sem0)
        gwait(rows1_v, sem1)
        scat(j0 + 1, rows1_v)
        return carry

    lax.fori_loop(0, (NCH - 2) // 2, pair, 0)
    gather(NCH - 1, rows1_v, sem1)
    gwait(rows0_v, sem0)
    scat(NCH - 2, rows0_v)
    gwait(rows1_v, sem1)
    scat(NCH - 1, rows1_v)
    plsc.subcore_barrier()
    pltpu.sync_copy(acc_sh.at[pl.ds(s * RPT, RPT)],
                    out_hbm.at[c, pl.ds(s * RPT, RPT)])


def _tc1_body(degp_ref, x_ref, w1_ref, g1_ref, dinv_ref):
    deg = degp_ref[0, :] + degp_ref[1, :] + 1.0
    dinv = lax.rsqrt(deg)
    h = jnp.dot(x_ref[...], w1_ref[...], preferred_element_type=jnp.float32)
    g1_ref[...] = h * dinv[:, None]
    dinv_ref[...] = dinv[:, None]


def _tc2_body(s_ref, g1_ref, dinv_ref, b1_ref, g2_ref):
    dinv = dinv_ref[...]
    agg = (s_ref[0] + s_ref[1] + g1_ref[...]) * dinv + b1_ref[...]
    r = jnp.maximum(agg, 0.0)
    g2_ref[...] = r * dinv


def _tc3_body(t_ref, g2_ref, dinv_ref, w2_ref, b2_ref, out_ref):
    agg = (t_ref[0] + t_ref[1] + g2_ref[...]) * dinv_ref[...]
    h2 = jnp.dot(agg, w2_ref[...], preferred_element_type=jnp.float32)
    h2 = h2 + b2_ref[...]
    m = jnp.max(h2, axis=1, keepdims=True)
    z = h2 - m
    lse = jnp.log(jnp.sum(jnp.exp(z), axis=1, keepdims=True))
    out_ref[...] = z - lse


_tc1 = pl.pallas_call(
    _tc1_body,
    out_shape=(jax.ShapeDtypeStruct((NP, H), jnp.float32),
               jax.ShapeDtypeStruct((NP, 1), jnp.float32)),
)
_tc2 = pl.pallas_call(
    _tc2_body,
    out_shape=jax.ShapeDtypeStruct((NP, H), jnp.float32),
)
_tc3 = pl.pallas_call(
    _tc3_body,
    out_shape=jax.ShapeDtypeStruct((NP, C), jnp.float32),
)


def kernel(x, edge_index, W1, b1, W2, b2):
    ei = edge_index.astype(jnp.int32)
    ei = jnp.pad(ei, ((0, 0), (0, EP - E)), constant_values=N)
    src = ei[0].reshape(NW, NCH, K)
    dst = ei[1].reshape(NW, NCH, K)
    xp = jnp.pad(x, ((0, NP - N), (0, 0)))
    degp = _deg_kernel(dst)
    g1, dinv = _tc1(degp, xp, W1)
    s1 = _scatter_kernel(g1, src, dst)
    g2 = _tc2(s1, g1, dinv, b1.reshape(1, H))
    s2 = _scatter_kernel(g2, src, dst)
    outp = _tc3(s2, g2, dinv, W2, b2.reshape(1, C))
    return outp[:N]


# speedup vs baseline: 1.0497x; 1.0497x over previous
import functools

import jax
import jax.numpy as jnp
from jax import lax
from jax.experimental import pallas as pl
from jax.experimental.pallas import tpu as pltpu
from jax.experimental.pallas import tpu_sc as plsc

N = 10000
D = 128
H = 16
C = 40
E = 320000
NP = 10240
NC = 2
NS = 16
NW = NC * NS
EW = E // NW
K = 80
NCH = EW // K
RPT = NP // NS

_mesh = plsc.VectorSubcoreMesh(core_axis_name="c", subcore_axis_name="s")


@functools.partial(
    pl.kernel,
    out_type=jax.ShapeDtypeStruct((NC, NP), jnp.float32),
    mesh=_mesh,
    scratch_types=[
        pltpu.VMEM((NCH, K), jnp.int32),
        pltpu.VMEM((K,), jnp.float32),
        pltpu.VMEM((RPT,), jnp.float32),
        pltpu.VMEM_SHARED((NP,), jnp.float32),
    ],
)
def _deg_kernel(dst_hbm, out_hbm, didx_v, ones_v, zero_v, deg_sh):
    c = lax.axis_index("c")
    s = lax.axis_index("s")
    w = c * NS + s
    for i in range(K // 16):
        ones_v[pl.ds(i * 16, 16)] = jnp.ones((16,), jnp.float32)
    for i in range(RPT // 16):
        zero_v[pl.ds(i * 16, 16)] = jnp.zeros((16,), jnp.float32)
    pltpu.sync_copy(zero_v, deg_sh.at[pl.ds(s * RPT, RPT)])
    plsc.subcore_barrier()
    pltpu.sync_copy(dst_hbm.at[w], didx_v)

    def chunk(j, carry):
        pltpu.sync_copy(ones_v, deg_sh.at[didx_v.at[j]], add=True)
        return carry

    lax.fori_loop(0, NCH, chunk, 0)
    plsc.subcore_barrier()
    pltpu.sync_copy(deg_sh.at[pl.ds(s * RPT, RPT)],
                    out_hbm.at[c, pl.ds(s * RPT, RPT)])


@functools.partial(
    pl.kernel,
    out_type=jax.ShapeDtypeStruct((NC, NP, H), jnp.float32),
    mesh=_mesh,
    scratch_types=[
        pltpu.VMEM((NCH, K), jnp.int32),
        pltpu.VMEM((NCH, K), jnp.int32),
        pltpu.VMEM((K, H), jnp.float32),
        pltpu.VMEM((K, H), jnp.float32),
        pltpu.VMEM_SHARED((NP, H), jnp.float32),
        pltpu.SemaphoreType.DMA,
        pltpu.SemaphoreType.DMA,
    ],
    compiler_params=pltpu.CompilerParams(use_tc_tiling_on_sc=False),
)
def _scatter_kernel(g_hbm, src_hbm, dst_hbm, out_hbm, sidx_v, didx_v, rows0_v,
                    rows1_v, acc_sh, sem0, sem1):
    c = lax.axis_index("c")
    s = lax.axis_index("s")
    w = c * NS + s
    for i in range(K):
        rows0_v[i, :] = jnp.zeros((H,), jnp.float32)

    for t in range(RPT // K):
        pltpu.sync_copy(rows0_v, acc_sh.at[pl.ds(s * RPT + t * K, K)])
    plsc.subcore_barrier()
    pltpu.sync_copy(src_hbm.at[w], sidx_v)
    pltpu.sync_copy(dst_hbm.at[w], didx_v)

    def gather(j, rows, sem):
        pltpu.async_copy(g_hbm.at[sidx_v.at[j]], rows, sem)

    def gwait(rows, sem):
        pltpu.make_async_copy(g_hbm.at[sidx_v.at[0]], rows, sem).wait()

    def scat(j, rows):
        pltpu.sync_copy(rows, acc_sh.at[didx_v.at[j]], add=True)

    gather(0, rows0_v, sem0)

    def pair(i, carry):
        j0 = 2 * i
        gather(j0 + 1, rows1_v, sem1)
        gwait(rows0_v, sem0)
        scat(j0, rows0_v)
        gather(j0 + 2, rows0_v, sem0)
        gwait(rows1_v, sem1)
        scat(j0 + 1, rows1_v)
        return carry

    lax.fori_loop(0, NCH // 2, pair, 0)
    gwait(rows0_v, sem0)
    scat(NCH - 1, rows0_v)
    plsc.subcore_barrier()
    pltpu.sync_copy(acc_sh.at[pl.ds(s * RPT, RPT)],
                    out_hbm.at[c, pl.ds(s * RPT, RPT)])


def _tc0_body(x_ref, w1_ref, h_ref):
    h = jnp.dot(x_ref[...], w1_ref[...], preferred_element_type=jnp.float32)
    h_ref[0:N, :] = h
    h_ref[N:NP, :] = jnp.zeros((NP - N, H), jnp.float32)


def _tc1_body(degp_ref, h_ref, g1_ref, dinv_ref):
    deg = degp_ref[0, :] + degp_ref[1, :] + 1.0
    dinv = lax.rsqrt(deg)
    g1_ref[...] = h_ref[...] * dinv[:, None]
    dinv_ref[...] = dinv[:, None]


def _tc2_body(s_ref, g1_ref, dinv_ref, b1_ref, g2_ref):
    dinv = dinv_ref[...]
    agg = (s_ref[0] + s_ref[1] + g1_ref[...]) * dinv + b1_ref[...]
    r = jnp.maximum(agg, 0.0)
    g2_ref[...] = r * dinv


def _tc3_body(t_ref, g2_ref, dinv_ref, w2_ref, b2_ref, out_ref):
    agg = (t_ref[0] + t_ref[1] + g2_ref[...]) * dinv_ref[...]
    h2 = jnp.dot(agg, w2_ref[...], preferred_element_type=jnp.float32)
    h2 = h2 + b2_ref[...]
    m = jnp.max(h2, axis=1, keepdims=True)
    z = h2 - m
    lse = jnp.log(jnp.sum(jnp.exp(z), axis=1, keepdims=True))
    out_ref[...] = z - lse


_tc0 = pl.pallas_call(
    _tc0_body,
    out_shape=jax.ShapeDtypeStruct((NP, H), jnp.float32),
)
_tc1 = pl.pallas_call(
    _tc1_body,
    out_shape=(jax.ShapeDtypeStruct((NP, H), jnp.float32),
               jax.ShapeDtypeStruct((NP, 1), jnp.float32)),
)
_tc2 = pl.pallas_call(
    _tc2_body,
    out_shape=jax.ShapeDtypeStruct((NP, H), jnp.float32),
)
_tc3 = pl.pallas_call(
    _tc3_body,
    out_shape=jax.ShapeDtypeStruct((NP, C), jnp.float32),
)


def kernel(x, edge_index, W1, b1, W2, b2):
    ei = edge_index.astype(jnp.int32)
    src = ei[0].reshape(NW, NCH, K)
    dst = ei[1].reshape(NW, NCH, K)
    h = _tc0(x, W1)
    degp = _deg_kernel(dst)
    g1, dinv = _tc1(degp, h)
    s1 = _scatter_kernel(g1, src, dst)
    g2 = _tc2(s1, g1, dinv, b1.reshape(1, H))
    s2 = _scatter_kernel(g2, src, dst)
    outp = _tc3(s2, g2, dinv, W2, b2.reshape(1, C))
    return outp[:N]


# speedup vs baseline: 1.4245x; 1.3570x over previous
import functools

import jax
import jax.numpy as jnp
from jax import lax
from jax.experimental import pallas as pl
from jax.experimental.pallas import tpu as pltpu
from jax.experimental.pallas import tpu_sc as plsc

N = 10000
D = 128
H = 16
C = 40
E = 320000
NP = 10240
NC = 2
NS = 16
NW = NC * NS
EW = E // NW
K = 80
NCH = EW // K
RPT = NP // NS

_mesh = plsc.VectorSubcoreMesh(core_axis_name="c", subcore_axis_name="s")


@functools.partial(
    pl.kernel,
    out_type=jax.ShapeDtypeStruct((NC, NP), jnp.float32),
    mesh=_mesh,
    scratch_types=[
        pltpu.VMEM((NCH, K), jnp.int32),
        pltpu.VMEM((K,), jnp.float32),
        pltpu.VMEM((RPT,), jnp.float32),
        pltpu.VMEM_SHARED((NP,), jnp.float32),
    ],
)
def _deg_kernel(dst_hbm, out_hbm, didx_v, ones_v, zero_v, deg_sh):
    c = lax.axis_index("c")
    s = lax.axis_index("s")
    w = c * NS + s
    for i in range(K // 16):
        ones_v[pl.ds(i * 16, 16)] = jnp.ones((16,), jnp.float32)
    for i in range(RPT // 16):
        zero_v[pl.ds(i * 16, 16)] = jnp.zeros((16,), jnp.float32)
    pltpu.sync_copy(zero_v, deg_sh.at[pl.ds(s * RPT, RPT)])
    plsc.subcore_barrier()
    pltpu.sync_copy(dst_hbm.at[w], didx_v)

    def chunk(j, carry):
        pltpu.sync_copy(ones_v, deg_sh.at[didx_v.at[j]], add=True)
        return carry

    lax.fori_loop(0, NCH, chunk, 0)
    plsc.subcore_barrier()
    pltpu.sync_copy(deg_sh.at[pl.ds(s * RPT, RPT)],
                    out_hbm.at[c, pl.ds(s * RPT, RPT)])


@functools.partial(
    pl.kernel,
    out_type=jax.ShapeDtypeStruct((NC, NP, H), jnp.float32),
    mesh=_mesh,
    scratch_types=[
        pltpu.VMEM((NCH, K), jnp.int32),
        pltpu.VMEM((NCH, K), jnp.int32),
        pltpu.VMEM((K, H), jnp.float32),
        pltpu.VMEM((K, H), jnp.float32),
        pltpu.VMEM_SHARED((NP, H), jnp.float32),
        pltpu.VMEM_SHARED((NP, H), jnp.float32),
        pltpu.SemaphoreType.DMA,
        pltpu.SemaphoreType.DMA,
    ],
    compiler_params=pltpu.CompilerParams(use_tc_tiling_on_sc=False),
)
def _scatter_kernel(g_hbm, src_hbm, dst_hbm, out_hbm, sidx_v, didx_v, rows0_v,
                    rows1_v, acc_sh, g_sh, sem0, sem1):
    c = lax.axis_index("c")
    s = lax.axis_index("s")
    w = c * NS + s
    for i in range(K):
        rows0_v[i, :] = jnp.zeros((H,), jnp.float32)

    pltpu.sync_copy(g_hbm.at[pl.ds(s * RPT, RPT)],
                    g_sh.at[pl.ds(s * RPT, RPT)])
    for t in range(RPT // K):
        pltpu.sync_copy(rows0_v, acc_sh.at[pl.ds(s * RPT + t * K, K)])
    plsc.subcore_barrier()
    pltpu.sync_copy(src_hbm.at[w], sidx_v)
    pltpu.sync_copy(dst_hbm.at[w], didx_v)

    def gather(j, rows, sem):
        pltpu.async_copy(g_sh.at[sidx_v.at[j]], rows, sem)

    def gwait(rows, sem):
        pltpu.make_async_copy(g_sh.at[sidx_v.at[0]], rows, sem).wait()

    def scat(j, rows):
        pltpu.sync_copy(rows, acc_sh.at[didx_v.at[j]], add=True)

    gather(0, rows0_v, sem0)

    def pair(i, carry):
        j0 = 2 * i
        gather(j0 + 1, rows1_v, sem1)
        gwait(rows0_v, sem0)
        scat(j0, rows0_v)
        gather(j0 + 2, rows0_v, sem0)
        gwait(rows1_v, sem1)
        scat(j0 + 1, rows1_v)
        return carry

    lax.fori_loop(0, NCH // 2, pair, 0)
    gwait(rows0_v, sem0)
    scat(NCH - 1, rows0_v)
    plsc.subcore_barrier()
    pltpu.sync_copy(acc_sh.at[pl.ds(s * RPT, RPT)],
                    out_hbm.at[c, pl.ds(s * RPT, RPT)])


def _tc0_body(x_ref, w1_ref, h_ref):
    h = jnp.dot(x_ref[...], w1_ref[...], preferred_element_type=jnp.float32)
    h_ref[0:N, :] = h
    h_ref[N:NP, :] = jnp.zeros((NP - N, H), jnp.float32)


def _tc1_body(degp_ref, h_ref, g1_ref, dinv_ref):
    deg = degp_ref[0, :] + degp_ref[1, :] + 1.0
    dinv = lax.rsqrt(deg)
    g1_ref[...] = h_ref[...] * dinv[:, None]
    dinv_ref[...] = dinv[:, None]


def _tc2_body(s_ref, g1_ref, dinv_ref, b1_ref, g2_ref):
    dinv = dinv_ref[...]
    agg = (s_ref[0] + s_ref[1] + g1_ref[...]) * dinv + b1_ref[...]
    r = jnp.maximum(agg, 0.0)
    g2_ref[...] = r * dinv


def _tc3_body(t_ref, g2_ref, dinv_ref, w2_ref, b2_ref, out_ref):
    agg = (t_ref[0] + t_ref[1] + g2_ref[...]) * dinv_ref[...]
    h2 = jnp.dot(agg, w2_ref[...], preferred_element_type=jnp.float32)
    h2 = h2 + b2_ref[...]
    m = jnp.max(h2, axis=1, keepdims=True)
    z = h2 - m
    lse = jnp.log(jnp.sum(jnp.exp(z), axis=1, keepdims=True))
    out_ref[...] = z - lse


_tc0 = pl.pallas_call(
    _tc0_body,
    out_shape=jax.ShapeDtypeStruct((NP, H), jnp.float32),
)
_tc1 = pl.pallas_call(
    _tc1_body,
    out_shape=(jax.ShapeDtypeStruct((NP, H), jnp.float32),
               jax.ShapeDtypeStruct((NP, 1), jnp.float32)),
)
_tc2 = pl.pallas_call(
    _tc2_body,
    out_shape=jax.ShapeDtypeStruct((NP, H), jnp.float32),
)
_tc3 = pl.pallas_call(
    _tc3_body,
    out_shape=jax.ShapeDtypeStruct((NP, C), jnp.float32),
)


def kernel(x, edge_index, W1, b1, W2, b2):
    ei = edge_index.astype(jnp.int32)
    src = ei[0].reshape(NW, NCH, K)
    dst = ei[1].reshape(NW, NCH, K)
    h = _tc0(x, W1)
    degp = _deg_kernel(dst)
    g1, dinv = _tc1(degp, h)
    s1 = _scatter_kernel(g1, src, dst)
    g2 = _tc2(s1, g1, dinv, b1.reshape(1, H))
    s2 = _scatter_kernel(g2, src, dst)
    outp = _tc3(s2, g2, dinv, W2, b2.reshape(1, C))
    return outp[:N]


# speedup vs baseline: 1.5270x; 1.0720x over previous
import functools

import jax
import jax.numpy as jnp
from jax import lax
from jax.experimental import pallas as pl
from jax.experimental.pallas import tpu as pltpu
from jax.experimental.pallas import tpu_sc as plsc

N = 10000
D = 128
H = 16
C = 40
E = 320000
NP = 10240
NC = 2
NS = 16
NW = NC * NS
EW = E // NW
K = 80
NCH = EW // K
RPT = NP // NS

_mesh = plsc.VectorSubcoreMesh(core_axis_name="c", subcore_axis_name="s")


@functools.partial(
    pl.kernel,
    out_type=jax.ShapeDtypeStruct((NC, NP), jnp.float32),
    mesh=_mesh,
    scratch_types=[
        pltpu.VMEM((NCH, K), jnp.int32),
        pltpu.VMEM((K,), jnp.float32),
        pltpu.VMEM((RPT,), jnp.float32),
        pltpu.VMEM_SHARED((NP,), jnp.float32),
    ],
)
def _deg_kernel(dst_hbm, out_hbm, didx_v, ones_v, zero_v, deg_sh):
    c = lax.axis_index("c")
    s = lax.axis_index("s")
    w = c * NS + s
    for i in range(K // 16):
        ones_v[pl.ds(i * 16, 16)] = jnp.ones((16,), jnp.float32)
    for i in range(RPT // 16):
        zero_v[pl.ds(i * 16, 16)] = jnp.zeros((16,), jnp.float32)
    pltpu.sync_copy(zero_v, deg_sh.at[pl.ds(s * RPT, RPT)])
    plsc.subcore_barrier()
    pltpu.sync_copy(dst_hbm.at[w], didx_v)

    def chunk(j, carry):
        pltpu.sync_copy(ones_v, deg_sh.at[didx_v.at[j]], add=True)
        return carry

    lax.fori_loop(0, NCH, chunk, 0)
    plsc.subcore_barrier()
    pltpu.sync_copy(deg_sh.at[pl.ds(s * RPT, RPT)],
                    out_hbm.at[c, pl.ds(s * RPT, RPT)])


def _pipeline(g_sh, acc_sh, sidx_v, didx_v, rows0_v, rows1_v, sem0, sem1):
    def gather(j, rows, sem):
        pltpu.async_copy(g_sh.at[sidx_v.at[j]], rows, sem)

    def gwait(rows, sem):
        pltpu.make_async_copy(g_sh.at[sidx_v.at[0]], rows, sem).wait()

    def scat(j, rows):
        pltpu.sync_copy(rows, acc_sh.at[didx_v.at[j]], add=True)

    gather(0, rows0_v, sem0)

    def pair(i, carry):
        j0 = 2 * i
        gather(j0 + 1, rows1_v, sem1)
        gwait(rows0_v, sem0)
        scat(j0, rows0_v)
        gather(j0 + 2, rows0_v, sem0)
        gwait(rows1_v, sem1)
        scat(j0 + 1, rows1_v)
        return carry

    lax.fori_loop(0, NCH // 2, pair, 0)
    gwait(rows0_v, sem0)
    scat(NCH - 1, rows0_v)


_SCAT_SCRATCH = [
    pltpu.VMEM((NCH, K), jnp.int32),
    pltpu.VMEM((NCH, K), jnp.int32),
    pltpu.VMEM((K, H), jnp.float32),
    pltpu.VMEM((K, H), jnp.float32),
    pltpu.VMEM((RPT, H), jnp.float32),
    pltpu.VMEM((RPT,), jnp.float32),
    pltpu.VMEM_SHARED((NP, H), jnp.float32),
    pltpu.VMEM_SHARED((NP, H), jnp.float32),
    pltpu.SemaphoreType.DMA,
    pltpu.SemaphoreType.DMA,
]


@functools.partial(
    pl.kernel,
    out_type=jax.ShapeDtypeStruct((NC, NP, H), jnp.float32),
    mesh=_mesh,
    scratch_types=_SCAT_SCRATCH,
    compiler_params=pltpu.CompilerParams(use_tc_tiling_on_sc=False),
)
def _scatter1_kernel(h_hbm, dinv_hbm, src_hbm, dst_hbm, out_hbm, sidx_v,
                     didx_v, rows0_v, rows1_v, sl_v, dinv_v, acc_sh, g_sh,
                     sem0, sem1):
    c = lax.axis_index("c")
    s = lax.axis_index("s")
    w = c * NS + s
    for i in range(K):
        rows0_v[i, :] = jnp.zeros((H,), jnp.float32)

    pltpu.sync_copy(h_hbm.at[pl.ds(s * RPT, RPT)], sl_v)
    pltpu.sync_copy(dinv_hbm.at[pl.ds(s * RPT, RPT)], dinv_v)

    def scale(r, carry):
        sl_v[r, :] = sl_v[r, :] * dinv_v[pl.ds(r, 1)][0]
        return carry

    lax.fori_loop(0, RPT, scale, 0)
    pltpu.sync_copy(sl_v, g_sh.at[pl.ds(s * RPT, RPT)])
    for t in range(RPT // K):
        pltpu.sync_copy(rows0_v, acc_sh.at[pl.ds(s * RPT + t * K, K)])
    plsc.subcore_barrier()
    pltpu.sync_copy(src_hbm.at[w], sidx_v)
    pltpu.sync_copy(dst_hbm.at[w], didx_v)
    _pipeline(g_sh, acc_sh, sidx_v, didx_v, rows0_v, rows1_v, sem0, sem1)
    plsc.subcore_barrier()
    pltpu.sync_copy(acc_sh.at[pl.ds(s * RPT, RPT)],
                    out_hbm.at[c, pl.ds(s * RPT, RPT)])


@functools.partial(
    pl.kernel,
    out_type=(jax.ShapeDtypeStruct((NC, NP, H), jnp.float32),
              jax.ShapeDtypeStruct((NP, H), jnp.float32)),
    mesh=_mesh,
    scratch_types=_SCAT_SCRATCH + [
        pltpu.VMEM((RPT, H), jnp.float32),
        pltpu.VMEM((RPT, H), jnp.float32),
        pltpu.VMEM((16,), jnp.float32),
    ],
    compiler_params=pltpu.CompilerParams(use_tc_tiling_on_sc=False),
)
def _scatter2_kernel(s1_hbm, h_hbm, dinv_hbm, b1_hbm, src_hbm, dst_hbm,
                     out_hbm, g2_hbm, sidx_v, didx_v, rows0_v, rows1_v, sl_v,
                     dinv_v, acc_sh, g_sh, sem0, sem1, p0_v, p1_v, b1_v):
    c = lax.axis_index("c")
    s = lax.axis_index("s")
    w = c * NS + s
    for i in range(K):
        rows0_v[i, :] = jnp.zeros((H,), jnp.float32)

    pltpu.sync_copy(h_hbm.at[pl.ds(s * RPT, RPT)], sl_v)
    pltpu.sync_copy(dinv_hbm.at[pl.ds(s * RPT, RPT)], dinv_v)
    pltpu.sync_copy(s1_hbm.at[0, pl.ds(s * RPT, RPT)], p0_v)
    pltpu.sync_copy(s1_hbm.at[1, pl.ds(s * RPT, RPT)], p1_v)
    pltpu.sync_copy(b1_hbm, b1_v)

    def stage(r, carry):
        dv = dinv_v[pl.ds(r, 1)][0]
        agg = (p0_v[r, :] + p1_v[r, :] + sl_v[r, :] * dv) * dv + b1_v[:]
        sl_v[r, :] = jnp.maximum(agg, 0.0) * dv
        return carry

    lax.fori_loop(0, RPT, stage, 0)
    pltpu.sync_copy(sl_v, g_sh.at[pl.ds(s * RPT, RPT)])
    pltpu.sync_copy(sl_v.at[pl.ds(c * (RPT // 2), RPT // 2)],
                    g2_hbm.at[pl.ds(s * RPT + c * (RPT // 2), RPT // 2)])
    for t in range(RPT // K):
        pltpu.sync_copy(rows0_v, acc_sh.at[pl.ds(s * RPT + t * K, K)])
    plsc.subcore_barrier()
    pltpu.sync_copy(src_hbm.at[w], sidx_v)
    pltpu.sync_copy(dst_hbm.at[w], didx_v)
    _pipeline(g_sh, acc_sh, sidx_v, didx_v, rows0_v, rows1_v, sem0, sem1)
    plsc.subcore_barrier()
    pltpu.sync_copy(acc_sh.at[pl.ds(s * RPT, RPT)],
                    out_hbm.at[c, pl.ds(s * RPT, RPT)])


def _tc0_body(x_ref, w1_ref, h_ref):
    h = jnp.dot(x_ref[...], w1_ref[...], preferred_element_type=jnp.float32)
    h_ref[0:N, :] = h
    h_ref[N:NP, :] = jnp.zeros((NP - N, H), jnp.float32)


def _tc1_body(degp_ref, dinvf_ref, dinv2_ref):
    deg = degp_ref[0, :] + degp_ref[1, :] + 1.0
    dinv = lax.rsqrt(deg)
    dinvf_ref[...] = dinv
    dinv2_ref[...] = dinv[:, None]


def _tc3_body(t_ref, g2_ref, dinv_ref, w2_ref, b2_ref, out_ref):
    agg = (t_ref[0] + t_ref[1] + g2_ref[...]) * dinv_ref[...]
    h2 = jnp.dot(agg, w2_ref[...], preferred_element_type=jnp.float32)
    h2 = h2[0:N, :] + b2_ref[...]
    m = jnp.max(h2, axis=1, keepdims=True)
    z = h2 - m
    lse = jnp.log(jnp.sum(jnp.exp(z), axis=1, keepdims=True))
    out_ref[...] = z - lse


_tc0 = pl.pallas_call(
    _tc0_body,
    out_shape=jax.ShapeDtypeStruct((NP, H), jnp.float32),
)
_tc1 = pl.pallas_call(
    _tc1_body,
    out_shape=(jax.ShapeDtypeStruct((NP,), jnp.float32),
               jax.ShapeDtypeStruct((NP, 1), jnp.float32)),
)
_tc3 = pl.pallas_call(
    _tc3_body,
    out_shape=jax.ShapeDtypeStruct((N, C), jnp.float32),
)


def kernel(x, edge_index, W1, b1, W2, b2):
    ei = edge_index.astype(jnp.int32)
    src = ei[0].reshape(NW, NCH, K)
    dst = ei[1].reshape(NW, NCH, K)
    h = _tc0(x, W1)
    degp = _deg_kernel(dst)
    dinvf, dinv2 = _tc1(degp)
    s1 = _scatter1_kernel(h, dinvf, src, dst)
    s2, g2 = _scatter2_kernel(s1, h, dinvf, b1, src, dst)
    return _tc3(s2, g2, dinv2, W2, b2.reshape(1, C))


# speedup vs baseline: 1.6878x; 1.1053x over previous
import functools

import jax
import jax.numpy as jnp
from jax import lax
from jax.experimental import pallas as pl
from jax.experimental.pallas import tpu as pltpu
from jax.experimental.pallas import tpu_sc as plsc

N = 10000
D = 128
H = 16
C = 40
E = 320000
NP = 10240
NC = 2
NS = 16
NW = NC * NS
EW = E // NW
K = 80
NCH = EW // K
RPT = NP // NS

_mesh = plsc.VectorSubcoreMesh(core_axis_name="c", subcore_axis_name="s")


@functools.partial(
    pl.kernel,
    out_type=jax.ShapeDtypeStruct((NC, NP), jnp.float32),
    mesh=_mesh,
    scratch_types=[
        pltpu.VMEM((NCH, K), jnp.int32),
        pltpu.VMEM((K,), jnp.float32),
        pltpu.VMEM((RPT,), jnp.float32),
        pltpu.VMEM_SHARED((NP,), jnp.float32),
        pltpu.SemaphoreType.DMA,
        pltpu.SemaphoreType.DMA,
        pltpu.SemaphoreType.DMA,
        pltpu.SemaphoreType.DMA,
    ],
)
def _deg_kernel(dst_hbm, out_hbm, didx_v, ones_v, zero_v, deg_sh, dsem0,
                dsem1, dsem2, dsem3):
    c = lax.axis_index("c")
    s = lax.axis_index("s")
    w = c * NS + s
    for i in range(K // 16):
        ones_v[pl.ds(i * 16, 16)] = jnp.ones((16,), jnp.float32)
    for i in range(RPT // 16):
        zero_v[pl.ds(i * 16, 16)] = jnp.zeros((16,), jnp.float32)
    pltpu.sync_copy(zero_v, deg_sh.at[pl.ds(s * RPT, RPT)])
    plsc.subcore_barrier()
    pltpu.sync_copy(dst_hbm.at[w], didx_v)

    def scat(j, sem):
        pltpu.async_copy(ones_v, deg_sh.at[didx_v.at[j]], sem, add=True)

    def swait(sem):
        pltpu.make_async_copy(ones_v, deg_sh.at[didx_v.at[0]], sem).wait()

    sems = [dsem0, dsem1, dsem2, dsem3]
    for r in range(4):
        scat(r, sems[r])

    def chunk(q, carry):
        j0 = 4 * q
        for r in range(4):
            swait(sems[r])
            scat(j0 + r, sems[r])
        return carry

    lax.fori_loop(1, NCH // 4, chunk, 0)
    for j in range(4 * (NCH // 4), NCH):
        swait(sems[j % 4])
        scat(j, sems[j % 4])
    for j in range(NCH, NCH + 4):
        swait(sems[j % 4])
    plsc.subcore_barrier()
    pltpu.sync_copy(deg_sh.at[pl.ds(s * RPT, RPT)],
                    out_hbm.at[c, pl.ds(s * RPT, RPT)])


def _pipeline(g_sh, acc_sh, sidx_v, didx_v, bufs, gsems, ssems):
    def gather(j, r):
        pltpu.async_copy(g_sh.at[sidx_v.at[j]], bufs[r], gsems[r])

    def gwait(r):
        pltpu.make_async_copy(g_sh.at[sidx_v.at[0]], bufs[r], gsems[r]).wait()

    def scat(j, r):
        pltpu.async_copy(bufs[r], acc_sh.at[didx_v.at[j]], ssems[r], add=True)

    def swait(r):
        pltpu.make_async_copy(bufs[r], acc_sh.at[didx_v.at[0]],
                              ssems[r]).wait()

    gather(0, 0)
    gather(1, 1)
    for j in range(4):
        gwait(j)
        scat(j, j)
        if j + 2 < 4:
            gather(j + 2, j + 2)
        else:
            swait((j + 2) % 4)
            gather(j + 2, (j + 2) % 4)

    def body(q, carry):
        j0 = 4 * q
        for r in range(4):
            gwait(r)
            scat(j0 + r, r)
            swait((r + 2) % 4)
            gather(j0 + r + 2, (r + 2) % 4)
        return carry

    _Q = NCH // 4 - 1
    lax.fori_loop(1, _Q, body, 0)
    for j in range(4 * _Q, NCH):
        r = j % 4
        gwait(r)
        scat(j, r)
        if j + 2 < NCH:
            swait((r + 2) % 4)
            gather(j + 2, (r + 2) % 4)
    for j in range(NCH - 4, NCH):
        swait(j % 4)


_SCAT_SCRATCH = [
    pltpu.VMEM((NCH, K), jnp.int32),
    pltpu.VMEM((NCH, K), jnp.int32),
    pltpu.VMEM((K, H), jnp.float32),
    pltpu.VMEM((K, H), jnp.float32),
    pltpu.VMEM((K, H), jnp.float32),
    pltpu.VMEM((K, H), jnp.float32),
    pltpu.VMEM((RPT, H), jnp.float32),
    pltpu.VMEM((RPT,), jnp.float32),
    pltpu.VMEM_SHARED((NP, H), jnp.float32),
    pltpu.VMEM_SHARED((NP, H), jnp.float32),
] + [pltpu.SemaphoreType.DMA] * 8


@functools.partial(
    pl.kernel,
    out_type=jax.ShapeDtypeStruct((NC, NP, H), jnp.float32),
    mesh=_mesh,
    scratch_types=_SCAT_SCRATCH,
    compiler_params=pltpu.CompilerParams(use_tc_tiling_on_sc=False),
)
def _scatter1_kernel(h_hbm, dinv_hbm, src_hbm, dst_hbm, out_hbm, sidx_v,
                     didx_v, b0_v, b1_v, b2_v, b3_v, sl_v, dinv_v, acc_sh,
                     g_sh, gsem0, gsem1, gsem2, gsem3, ssem0, ssem1, ssem2,
                     ssem3):
    c = lax.axis_index("c")
    s = lax.axis_index("s")
    w = c * NS + s
    for i in range(K):
        b0_v[i, :] = jnp.zeros((H,), jnp.float32)

    pltpu.sync_copy(h_hbm.at[pl.ds(s * RPT, RPT)], sl_v)
    pltpu.sync_copy(dinv_hbm.at[pl.ds(s * RPT, RPT)], dinv_v)

    def scale(r, carry):
        sl_v[r, :] = sl_v[r, :] * dinv_v[pl.ds(r, 1)][0]
        return carry

    lax.fori_loop(0, RPT, scale, 0)
    pltpu.sync_copy(sl_v, g_sh.at[pl.ds(s * RPT, RPT)])
    for t in range(RPT // K):
        pltpu.sync_copy(b0_v, acc_sh.at[pl.ds(s * RPT + t * K, K)])
    plsc.subcore_barrier()
    pltpu.sync_copy(src_hbm.at[w], sidx_v)
    pltpu.sync_copy(dst_hbm.at[w], didx_v)
    _pipeline(g_sh, acc_sh, sidx_v, didx_v, [b0_v, b1_v, b2_v, b3_v],
              [gsem0, gsem1, gsem2, gsem3], [ssem0, ssem1, ssem2, ssem3])
    plsc.subcore_barrier()
    pltpu.sync_copy(acc_sh.at[pl.ds(s * RPT, RPT)],
                    out_hbm.at[c, pl.ds(s * RPT, RPT)])


@functools.partial(
    pl.kernel,
    out_type=(jax.ShapeDtypeStruct((NC, NP, H), jnp.float32),
              jax.ShapeDtypeStruct((NP, H), jnp.float32)),
    mesh=_mesh,
    scratch_types=_SCAT_SCRATCH + [
        pltpu.VMEM((RPT, H), jnp.float32),
        pltpu.VMEM((RPT, H), jnp.float32),
        pltpu.VMEM((16,), jnp.float32),
    ],
    compiler_params=pltpu.CompilerParams(use_tc_tiling_on_sc=False),
)
def _scatter2_kernel(s1_hbm, h_hbm, dinv_hbm, b1_hbm, src_hbm, dst_hbm,
                     out_hbm, g2_hbm, sidx_v, didx_v, b0_v, b1buf_v, b2_v,
                     b3_v, sl_v, dinv_v, acc_sh, g_sh, gsem0, gsem1, gsem2,
                     gsem3, ssem0, ssem1, ssem2, ssem3, p0_v, p1_v, b1_v):
    c = lax.axis_index("c")
    s = lax.axis_index("s")
    w = c * NS + s
    for i in range(K):
        b0_v[i, :] = jnp.zeros((H,), jnp.float32)

    pltpu.sync_copy(h_hbm.at[pl.ds(s * RPT, RPT)], sl_v)
    pltpu.sync_copy(dinv_hbm.at[pl.ds(s * RPT, RPT)], dinv_v)
    pltpu.sync_copy(s1_hbm.at[0, pl.ds(s * RPT, RPT)], p0_v)
    pltpu.sync_copy(s1_hbm.at[1, pl.ds(s * RPT, RPT)], p1_v)
    pltpu.sync_copy(b1_hbm, b1_v)

    def stage(r, carry):
        dv = dinv_v[pl.ds(r, 1)][0]
        agg = (p0_v[r, :] + p1_v[r, :] + sl_v[r, :] * dv) * dv + b1_v[:]
        sl_v[r, :] = jnp.maximum(agg, 0.0) * dv
        return carry

    lax.fori_loop(0, RPT, stage, 0)
    pltpu.sync_copy(sl_v, g_sh.at[pl.ds(s * RPT, RPT)])
    pltpu.sync_copy(sl_v.at[pl.ds(c * (RPT // 2), RPT // 2)],
                    g2_hbm.at[pl.ds(s * RPT + c * (RPT // 2), RPT // 2)])
    for t in range(RPT // K):
        pltpu.sync_copy(b0_v, acc_sh.at[pl.ds(s * RPT + t * K, K)])
    plsc.subcore_barrier()
    pltpu.sync_copy(src_hbm.at[w], sidx_v)
    pltpu.sync_copy(dst_hbm.at[w], didx_v)
    _pipeline(g_sh, acc_sh, sidx_v, didx_v, [b0_v, b1buf_v, b2_v, b3_v],
              [gsem0, gsem1, gsem2, gsem3], [ssem0, ssem1, ssem2, ssem3])
    plsc.subcore_barrier()
    pltpu.sync_copy(acc_sh.at[pl.ds(s * RPT, RPT)],
                    out_hbm.at[c, pl.ds(s * RPT, RPT)])


def _tc0_body(x_ref, w1_ref, h_ref):
    h = jnp.dot(x_ref[...], w1_ref[...], preferred_element_type=jnp.float32)
    h_ref[0:N, :] = h
    h_ref[N:NP, :] = jnp.zeros((NP - N, H), jnp.float32)


def _tc1_body(degp_ref, dinvf_ref, dinv2_ref):
    deg = degp_ref[0, :] + degp_ref[1, :] + 1.0
    dinv = lax.rsqrt(deg)
    dinvf_ref[...] = dinv
    dinv2_ref[...] = dinv[:, None]


def _tc3_body(t_ref, g2_ref, dinv_ref, w2_ref, b2_ref, out_ref):
    agg = (t_ref[0] + t_ref[1] + g2_ref[...]) * dinv_ref[...]
    h2 = jnp.dot(agg, w2_ref[...], preferred_element_type=jnp.float32)
    h2 = h2[0:N, :] + b2_ref[...]
    m = jnp.max(h2, axis=1, keepdims=True)
    z = h2 - m
    lse = jnp.log(jnp.sum(jnp.exp(z), axis=1, keepdims=True))
    out_ref[...] = z - lse


_tc0 = pl.pallas_call(
    _tc0_body,
    out_shape=jax.ShapeDtypeStruct((NP, H), jnp.float32),
)
_tc1 = pl.pallas_call(
    _tc1_body,
    out_shape=(jax.ShapeDtypeStruct((NP,), jnp.float32),
               jax.ShapeDtypeStruct((NP, 1), jnp.float32)),
)
_tc3 = pl.pallas_call(
    _tc3_body,
    out_shape=jax.ShapeDtypeStruct((N, C), jnp.float32),
)


def kernel(x, edge_index, W1, b1, W2, b2):
    ei = edge_index.astype(jnp.int32)
    src = ei[0].reshape(NW, NCH, K)
    dst = ei[1].reshape(NW, NCH, K)
    h = _tc0(x, W1)
    degp = _deg_kernel(dst)
    dinvf, dinv2 = _tc1(degp)
    s1 = _scatter1_kernel(h, dinvf, src, dst)
    s2, g2 = _scatter2_kernel(s1, h, dinvf, b1, src, dst)
    return _tc3(s2, g2, dinv2, W2, b2.reshape(1, C))


# speedup vs baseline: 1.7546x; 1.0396x over previous
import functools

import jax
import jax.numpy as jnp
from jax import lax
from jax.experimental import pallas as pl
from jax.experimental.pallas import tpu as pltpu
from jax.experimental.pallas import tpu_sc as plsc

N = 10000
D = 128
H = 16
C = 40
E = 320000
NP = 10240
NC = 2
NS = 16
NW = NC * NS
EW = E // NW
K = 80
NCH = EW // K
RPT = NP // NS

_mesh = plsc.VectorSubcoreMesh(core_axis_name="c", subcore_axis_name="s")


@functools.partial(
    pl.kernel,
    out_type=jax.ShapeDtypeStruct((NC, NP), jnp.float32),
    mesh=_mesh,
    scratch_types=[
        pltpu.VMEM((NCH, K), jnp.int32),
        pltpu.VMEM((K,), jnp.float32),
        pltpu.VMEM((RPT,), jnp.float32),
        pltpu.VMEM_SHARED((NP,), jnp.float32),
        pltpu.SemaphoreType.DMA,
        pltpu.SemaphoreType.DMA,
        pltpu.SemaphoreType.DMA,
        pltpu.SemaphoreType.DMA,
    ],
)
def _deg_kernel(dst_hbm, out_hbm, didx_v, ones_v, zero_v, deg_sh, dsem0,
                dsem1, dsem2, dsem3):
    c = lax.axis_index("c")
    s = lax.axis_index("s")
    w = c * NS + s
    for i in range(K // 16):
        ones_v[pl.ds(i * 16, 16)] = jnp.ones((16,), jnp.float32)
    for i in range(RPT // 16):
        zero_v[pl.ds(i * 16, 16)] = jnp.zeros((16,), jnp.float32)
    pltpu.sync_copy(zero_v, deg_sh.at[pl.ds(s * RPT, RPT)])
    plsc.subcore_barrier()
    pltpu.sync_copy(dst_hbm.at[w], didx_v)

    def scat(j, sem):
        pltpu.async_copy(ones_v, deg_sh.at[didx_v.at[j]], sem, add=True)

    def swait(sem):
        pltpu.make_async_copy(ones_v, deg_sh.at[didx_v.at[0]], sem).wait()

    sems = [dsem0, dsem1, dsem2, dsem3]
    for r in range(4):
        scat(r, sems[r])

    def chunk(q, carry):
        j0 = 4 * q
        for r in range(4):
            swait(sems[r])
            scat(j0 + r, sems[r])
        return carry

    lax.fori_loop(1, NCH // 4, chunk, 0)
    for j in range(4 * (NCH // 4), NCH):
        swait(sems[j % 4])
        scat(j, sems[j % 4])
    for j in range(NCH, NCH + 4):
        swait(sems[j % 4])
    plsc.subcore_barrier()
    pltpu.sync_copy(deg_sh.at[pl.ds(s * RPT, RPT)],
                    out_hbm.at[c, pl.ds(s * RPT, RPT)])


def _pipeline(g_sh, acc_sh, sidx_v, didx_v, bufs, gsems, ssems):
    def gather(j, r):
        pltpu.async_copy(g_sh.at[sidx_v.at[j]], bufs[r], gsems[r])

    def gwait(r):
        pltpu.make_async_copy(g_sh.at[sidx_v.at[0]], bufs[r], gsems[r]).wait()

    def scat(j, r):
        pltpu.async_copy(bufs[r], acc_sh.at[didx_v.at[j]], ssems[r], add=True)

    def swait(r):
        pltpu.make_async_copy(bufs[r], acc_sh.at[didx_v.at[0]],
                              ssems[r]).wait()

    gather(0, 0)
    gather(1, 1)
    for j in range(4):
        gwait(j)
        scat(j, j)
        if j + 2 < 4:
            gather(j + 2, j + 2)
        else:
            swait((j + 2) % 4)
            gather(j + 2, (j + 2) % 4)

    def body(q, carry):
        j0 = 4 * q
        for r in range(4):
            gwait(r)
            scat(j0 + r, r)
            swait((r + 2) % 4)
            gather(j0 + r + 2, (r + 2) % 4)
        return carry

    _Q = NCH // 4 - 1
    lax.fori_loop(1, _Q, body, 0)
    for j in range(4 * _Q, NCH):
        r = j % 4
        gwait(r)
        scat(j, r)
        if j + 2 < NCH:
            swait((r + 2) % 4)
            gather(j + 2, (r + 2) % 4)
    for j in range(NCH - 4, NCH):
        swait(j % 4)


_SCAT_SCRATCH = [
    pltpu.VMEM((NCH, K), jnp.int32),
    pltpu.VMEM((NCH, K), jnp.int32),
    pltpu.VMEM((K, H), jnp.float32),
    pltpu.VMEM((K, H), jnp.float32),
    pltpu.VMEM((K, H), jnp.float32),
    pltpu.VMEM((K, H), jnp.float32),
    pltpu.VMEM((RPT, H), jnp.float32),
    pltpu.VMEM((RPT,), jnp.float32),
    pltpu.VMEM_SHARED((NP, H), jnp.float32),
    pltpu.VMEM_SHARED((NP, H), jnp.float32),
] + [pltpu.SemaphoreType.DMA] * 8


@functools.partial(
    pl.kernel,
    out_type=jax.ShapeDtypeStruct((NC, NP, H), jnp.float32),
    mesh=_mesh,
    scratch_types=_SCAT_SCRATCH,
    compiler_params=pltpu.CompilerParams(use_tc_tiling_on_sc=False),
)
def _scatter1_kernel(h_hbm, dinv_hbm, src_hbm, dst_hbm, out_hbm, sidx_v,
                     didx_v, b0_v, b1_v, b2_v, b3_v, sl_v, dinv_v, acc_sh,
                     g_sh, gsem0, gsem1, gsem2, gsem3, ssem0, ssem1, ssem2,
                     ssem3):
    c = lax.axis_index("c")
    s = lax.axis_index("s")
    w = c * NS + s
    for i in range(K):
        b0_v[i, :] = jnp.zeros((H,), jnp.float32)

    pltpu.sync_copy(h_hbm.at[pl.ds(s * RPT, RPT)], sl_v)
    pltpu.sync_copy(dinv_hbm.at[pl.ds(s * RPT, RPT)], dinv_v)

    def scale(g, carry):
        base = g * 16
        dvg = dinv_v[pl.ds(base, 16)]
        for r in range(16):
            sl_v[base + r, :] = sl_v[base + r, :] * dvg[r]
        return carry

    lax.fori_loop(0, RPT // 16, scale, 0)
    pltpu.sync_copy(sl_v, g_sh.at[pl.ds(s * RPT, RPT)])
    for t in range(RPT // K):
        pltpu.sync_copy(b0_v, acc_sh.at[pl.ds(s * RPT + t * K, K)])
    plsc.subcore_barrier()
    pltpu.sync_copy(src_hbm.at[w], sidx_v)
    pltpu.sync_copy(dst_hbm.at[w], didx_v)
    _pipeline(g_sh, acc_sh, sidx_v, didx_v, [b0_v, b1_v, b2_v, b3_v],
              [gsem0, gsem1, gsem2, gsem3], [ssem0, ssem1, ssem2, ssem3])
    plsc.subcore_barrier()
    pltpu.sync_copy(acc_sh.at[pl.ds(s * RPT, RPT)],
                    out_hbm.at[c, pl.ds(s * RPT, RPT)])


@functools.partial(
    pl.kernel,
    out_type=(jax.ShapeDtypeStruct((NC, NP, H), jnp.float32),
              jax.ShapeDtypeStruct((NP, H), jnp.float32)),
    mesh=_mesh,
    scratch_types=_SCAT_SCRATCH + [
        pltpu.VMEM((RPT, H), jnp.float32),
        pltpu.VMEM((RPT, H), jnp.float32),
        pltpu.VMEM((16,), jnp.float32),
    ],
    compiler_params=pltpu.CompilerParams(use_tc_tiling_on_sc=False),
)
def _scatter2_kernel(s1_hbm, h_hbm, dinv_hbm, b1_hbm, src_hbm, dst_hbm,
                     out_hbm, g2_hbm, sidx_v, didx_v, b0_v, b1buf_v, b2_v,
                     b3_v, sl_v, dinv_v, acc_sh, g_sh, gsem0, gsem1, gsem2,
                     gsem3, ssem0, ssem1, ssem2, ssem3, p0_v, p1_v, b1_v):
    c = lax.axis_index("c")
    s = lax.axis_index("s")
    w = c * NS + s
    for i in range(K):
        b0_v[i, :] = jnp.zeros((H,), jnp.float32)

    pltpu.sync_copy(h_hbm.at[pl.ds(s * RPT, RPT)], sl_v)
    pltpu.sync_copy(dinv_hbm.at[pl.ds(s * RPT, RPT)], dinv_v)
    pltpu.sync_copy(s1_hbm.at[0, pl.ds(s * RPT, RPT)], p0_v)
    pltpu.sync_copy(s1_hbm.at[1, pl.ds(s * RPT, RPT)], p1_v)
    pltpu.sync_copy(b1_hbm, b1_v)

    def stage(g, carry):
        base = g * 16
        dvg = dinv_v[pl.ds(base, 16)]
        for r in range(16):
            dv = dvg[r]
            agg = (p0_v[base + r, :] + p1_v[base + r, :]
                   + sl_v[base + r, :] * dv) * dv + b1_v[:]
            sl_v[base + r, :] = jnp.maximum(agg, 0.0) * dv
        return carry

    lax.fori_loop(0, RPT // 16, stage, 0)
    pltpu.sync_copy(sl_v, g_sh.at[pl.ds(s * RPT, RPT)])
    pltpu.sync_copy(sl_v.at[pl.ds(c * (RPT // 2), RPT // 2)],
                    g2_hbm.at[pl.ds(s * RPT + c * (RPT // 2), RPT // 2)])
    for t in range(RPT // K):
        pltpu.sync_copy(b0_v, acc_sh.at[pl.ds(s * RPT + t * K, K)])
    plsc.subcore_barrier()
    pltpu.sync_copy(src_hbm.at[w], sidx_v)
    pltpu.sync_copy(dst_hbm.at[w], didx_v)
    _pipeline(g_sh, acc_sh, sidx_v, didx_v, [b0_v, b1buf_v, b2_v, b3_v],
              [gsem0, gsem1, gsem2, gsem3], [ssem0, ssem1, ssem2, ssem3])
    plsc.subcore_barrier()
    pltpu.sync_copy(acc_sh.at[pl.ds(s * RPT, RPT)],
                    out_hbm.at[c, pl.ds(s * RPT, RPT)])


def _tc0_body(x_ref, w1_ref, h_ref):
    h = jnp.dot(x_ref[...], w1_ref[...], preferred_element_type=jnp.float32)
    h_ref[0:N, :] = h
    h_ref[N:NP, :] = jnp.zeros((NP - N, H), jnp.float32)


def _tc1_body(degp_ref, dinvf_ref, dinv2_ref):
    deg = degp_ref[0, :] + degp_ref[1, :] + 1.0
    dinv = lax.rsqrt(deg)
    dinvf_ref[...] = dinv
    dinv2_ref[...] = dinv[:, None]


def _tc3_body(t_ref, g2_ref, dinv_ref, w2_ref, b2_ref, out_ref):
    agg = (t_ref[0] + t_ref[1] + g2_ref[...]) * dinv_ref[...]
    h2 = jnp.dot(agg, w2_ref[...], preferred_element_type=jnp.float32)
    h2 = h2[0:N, :] + b2_ref[...]
    m = jnp.max(h2, axis=1, keepdims=True)
    z = h2 - m
    lse = jnp.log(jnp.sum(jnp.exp(z), axis=1, keepdims=True))
    out_ref[...] = z - lse


_tc0 = pl.pallas_call(
    _tc0_body,
    out_shape=jax.ShapeDtypeStruct((NP, H), jnp.float32),
)
_tc1 = pl.pallas_call(
    _tc1_body,
    out_shape=(jax.ShapeDtypeStruct((NP,), jnp.float32),
               jax.ShapeDtypeStruct((NP, 1), jnp.float32)),
)
_tc3 = pl.pallas_call(
    _tc3_body,
    out_shape=jax.ShapeDtypeStruct((N, C), jnp.float32),
)


def kernel(x, edge_index, W1, b1, W2, b2):
    ei = edge_index.astype(jnp.int32)
    src = ei[0].reshape(NW, NCH, K)
    dst = ei[1].reshape(NW, NCH, K)
    h = _tc0(x, W1)
    degp = _deg_kernel(dst)
    dinvf, dinv2 = _tc1(degp)
    s1 = _scatter1_kernel(h, dinvf, src, dst)
    s2, g2 = _scatter2_kernel(s1, h, dinvf, b1, src, dst)
    return _tc3(s2, g2, dinv2, W2, b2.reshape(1, C))
